# Initial kernel scaffold; baseline (speedup 1.0000x reference)
import jax
import jax.numpy as jnp
from jax.experimental import pallas as pl


def kernel(x, edge_index, edge_attr, batch, interface_pos, graph_num, W0, attL0, eW0, eb0, b0, W1, attL1, eW1, eb1, b1, W2, attL2, eW2, eb2, b2, gate_w, lin1_W, lin1_b, lin2_W, lin2_b):
    raise NotImplementedError("write your pallas kernel here")


# speedup vs baseline: 12.3203x; 12.3203x over previous
import functools

import jax
import jax.numpy as jnp
from jax import lax
from jax.experimental import pallas as pl
from jax.experimental.pallas import tpu as pltpu
from jax.experimental.pallas import tpu_sc as plsc

N = 10000
NBLK = 79
NP = NBLK * 128
E0 = 160000
EE = E0 + N
R = 128
NB = 42
CHUNK = NB * R
NT = 32
EPAD = NT * CHUNK
DE = 4
C = 128
G = 16
NI = 2048
NI_W = NI // NT


def _pre_body(x_ref, w_ref, attl_ref, xl_ref, al_ref, m_ref):
    i = pl.program_id(0)
    xb = x_ref[...]
    xl = lax.dot_general(xb, w_ref[...], (((1,), (1,)), ((), ())),
                         preferred_element_type=jnp.float32)
    xl_ref[...] = xl
    al = jnp.sum(xl * attl_ref[...], axis=1)
    al_ref[0, 0, :] = al
    bm = jnp.max(al)
    lr = jnp.maximum(2.0 * bm, 0.4 * bm)

    @pl.when(i == 0)
    def _():
        m_ref[...] = jnp.full((8, 128), -jnp.inf, jnp.float32)

    m_ref[...] = jnp.maximum(m_ref[...], lr)


def _tc_pre(x, W, attl2):
    return pl.pallas_call(
        _pre_body,
        grid=(NBLK,),
        in_specs=[
            pl.BlockSpec((128, 128), lambda i: (i, 0)),
            pl.BlockSpec((128, 128), lambda i: (0, 0)),
            pl.BlockSpec((1, 128), lambda i: (0, 0)),
        ],
        out_specs=[
            pl.BlockSpec((128, 128), lambda i: (i, 0)),
            pl.BlockSpec((1, 1, 128), lambda i: (i, 0, 0)),
            pl.BlockSpec((8, 128), lambda i: (0, 0)),
        ],
        out_shape=[
            jax.ShapeDtypeStruct((NP, 128), jnp.float32),
            jax.ShapeDtypeStruct((NBLK, 1, 128), jnp.float32),
            jax.ShapeDtypeStruct((8, 128), jnp.float32),
        ],
    )(x, W, attl2)


def _sc_p1_body(src_h, dst_h, ea_h, al_h, m_h,
                sparts_h, tparts_h, e_out_h,
                al_v, m_v, src_v, dst_v, ea_v, e_v, s_v, t_v):
    cid = lax.axis_index("c")
    sid = lax.axis_index("s")
    w = cid * 16 + sid
    pltpu.sync_copy(al_h, al_v)
    pltpu.sync_copy(m_h.at[pl.ds(0, 16)], m_v)
    pltpu.sync_copy(src_h.at[w], src_v)
    pltpu.sync_copy(dst_h.at[w], dst_v)
    pltpu.sync_copy(ea_h.at[w], ea_v)

    zero16 = jnp.zeros((16,), jnp.float32)

    def zs(i, _):
        s_v[pl.ds(i * 16, 16)] = zero16
        return 0

    lax.fori_loop(0, NP // 16, zs, 0)

    def zt(i, _):
        t_v[pl.ds(i * 16, 16)] = zero16
        return 0

    lax.fori_loop(0, 4 * NP // 16, zt, 0)

    eb_base = w * CHUNK
    lane = lax.iota(jnp.int32, 16)
    mvec = m_v[...]

    def jbody(j, _):
        for k in range(8):
            off = j * R + k * 16
            sv = src_v[j, pl.ds(k * 16, 16)]
            dv = dst_v[j, pl.ds(k * 16, 16)]
            als = plsc.load_gather(al_v, [sv])
            ald = plsc.load_gather(al_v, [dv])
            z = als + ald
            lr2 = jnp.maximum(z, 0.2 * z)
            e = jnp.exp(lr2 - mvec)
            valid = (eb_base + off + lane) < EE
            e = jnp.where(valid, e, 0.0)
            e_v[pl.ds(off, 16)] = e
            plsc.addupdate_scatter(s_v, [dv], e)
            for c4 in range(4):
                raw = ea_v[pl.ds(c4 * CHUNK + off, 16)]
                eav = jnp.where(raw > 0.0, 1.0 / raw, 0.0)
                plsc.addupdate_scatter(t_v, [dv + c4 * NP], e * eav)
        return 0

    lax.fori_loop(0, NB, jbody, 0)
    pltpu.sync_copy(s_v, sparts_h.at[w])
    pltpu.sync_copy(t_v, tparts_h.at[w])
    pltpu.sync_copy(e_v, e_out_h.at[w])


def _sc_p1(srcp, dstp, eat, al_flat, m_flat):
    mesh = plsc.VectorSubcoreMesh(core_axis_name="c", subcore_axis_name="s",
                                  num_cores=2, num_subcores=16)
    f = pl.kernel(
        _sc_p1_body,
        out_type=[
            jax.ShapeDtypeStruct((NT, NP), jnp.float32),
            jax.ShapeDtypeStruct((NT, 4 * NP), jnp.float32),
            jax.ShapeDtypeStruct((NT, CHUNK), jnp.float32),
        ],
        mesh=mesh,
        compiler_params=pltpu.CompilerParams(needs_layout_passes=False),
        scratch_types=[
            pltpu.VMEM((NP,), jnp.float32),
            pltpu.VMEM((16,), jnp.float32),
            pltpu.VMEM((NB, R), jnp.int32),
            pltpu.VMEM((NB, R), jnp.int32),
            pltpu.VMEM((4 * CHUNK,), jnp.float32),
            pltpu.VMEM((CHUNK,), jnp.float32),
            pltpu.VMEM((NP,), jnp.float32),
            pltpu.VMEM((4 * NP,), jnp.float32),
        ],
    )
    return f(srcp, dstp, eat, al_flat, m_flat)


def _sc_p2_body(src_h, dst_h, e_h, xl_h, acc_h,
                src_v, dst_v, e_v, rows_v, acc_sh, sem):
    cid = lax.axis_index("c")
    sid = lax.axis_index("s")
    w = cid * 16 + sid
    pltpu.sync_copy(src_h.at[w], src_v)
    pltpu.sync_copy(dst_h.at[w], dst_v)
    pltpu.sync_copy(e_h.at[w], e_v)

    zero16 = jnp.zeros((16,), jnp.float32)

    def zr(r, _):
        for q in range(8):
            rows_v[r, pl.ds(q * 16, 16)] = zero16
        return 0

    lax.fori_loop(0, R, zr, 0)

    base = sid * (NP // 16)
    for q in range(4):
        pltpu.sync_copy(rows_v, acc_sh.at[pl.ds(base + q * 128, 128)])
    pltpu.sync_copy(rows_v.at[pl.ds(0, 120)],
                    acc_sh.at[pl.ds(base + 512, 120)])
    plsc.subcore_barrier()

    def jbody(j, _):
        pltpu.async_copy(xl_h.at[src_v.at[j]], rows_v, sem).wait()

        def rbody(r, _):
            er = plsc.load_gather(e_v, [jnp.zeros((16,), jnp.int32) + j * R + r])
            for q in range(8):
                rows_v[r, pl.ds(q * 16, 16)] = rows_v[r, pl.ds(q * 16, 16)] * er
            return 0

        lax.fori_loop(0, R, rbody, 0)
        pltpu.sync_copy(rows_v, acc_sh.at[dst_v.at[j]], add=True)
        return 0

    lax.fori_loop(0, NB, jbody, 0)
    plsc.subcore_barrier()
    nseg = NP // 16
    for q in range(4):
        pltpu.sync_copy(acc_sh.at[pl.ds(base + q * 128, 128)],
                        acc_h.at[cid].at[pl.ds(base + q * 128, 128)])
    pltpu.sync_copy(acc_sh.at[pl.ds(base + 512, nseg - 512)],
                    acc_h.at[cid].at[pl.ds(base + 512, nseg - 512)])


def _sc_p2(srcp, dstp, e_h, xl):
    mesh = plsc.VectorSubcoreMesh(core_axis_name="c", subcore_axis_name="s",
                                  num_cores=2, num_subcores=16)
    f = pl.kernel(
        _sc_p2_body,
        out_type=[
            jax.ShapeDtypeStruct((2, NP, 128), jnp.float32),
        ],
        mesh=mesh,
        compiler_params=pltpu.CompilerParams(needs_layout_passes=False),
        scratch_types=[
            pltpu.VMEM((NB, R), jnp.int32),
            pltpu.VMEM((NB, R), jnp.int32),
            pltpu.VMEM((CHUNK,), jnp.float32),
            pltpu.VMEM((R, 128), jnp.float32),
            pltpu.VMEM_SHARED((NP, 128), jnp.float32),
            pltpu.SemaphoreType.DMA,
        ],
    )
    (acc,) = f(srcp, dstp, e_h, xl)
    return acc


def _post_body(jk, *refs):
    if jk:
        (s_ref, t_ref, acc_ref, ew_ref, ebb_ref, x1_ref, x2_ref,
         out_ref, jk_ref) = refs
    else:
        s_ref, t_ref, acc_ref, ew_ref, ebb_ref, out_ref = refs
    s = jnp.sum(s_ref[...], axis=0)
    t = jnp.sum(t_ref[...], axis=0)
    acc = acc_ref[0] + acc_ref[1]
    ee = lax.dot_general(t, ew_ref[...], (((0,), (0,)), ((), ())),
                         preferred_element_type=jnp.float32)
    sc = jnp.maximum(s, 1e-30)[:, None]
    x3 = jnp.tanh((acc + ee) / sc + ebb_ref[...])
    out_ref[...] = x3
    if jk:
        jk_ref[...] = jnp.maximum(jnp.maximum(x1_ref[...], x2_ref[...]), x3)


def _tc_post(sparts, tparts3, acc, ewT, ebb, x1=None, x2=None):
    jk = x1 is not None
    in_specs = [
        pl.BlockSpec((NT, 128), lambda i: (0, i)),
        pl.BlockSpec((NT, 4, 128), lambda i: (0, 0, i)),
        pl.BlockSpec((2, 128, 128), lambda i: (0, i, 0)),
        pl.BlockSpec((4, 128), lambda i: (0, 0)),
        pl.BlockSpec((1, 128), lambda i: (0, 0)),
    ]
    out_specs = [pl.BlockSpec((128, 128), lambda i: (i, 0))]
    out_shape = [jax.ShapeDtypeStruct((NP, 128), jnp.float32)]
    args = [sparts, tparts3, acc, ewT, ebb]
    if jk:
        in_specs += [pl.BlockSpec((128, 128), lambda i: (i, 0))] * 2
        out_specs += [pl.BlockSpec((128, 128), lambda i: (i, 0))]
        out_shape += [jax.ShapeDtypeStruct((NP, 128), jnp.float32)]
        args += [x1, x2]
    res = pl.pallas_call(
        functools.partial(_post_body, jk),
        grid=(NBLK,),
        in_specs=in_specs,
        out_specs=out_specs,
        out_shape=out_shape,
    )(*args)
    return res


def _sc_gather_body(xjk_h, ip_h, batch_h, xi_h, seg_h,
                    batch_v, ip_v, rows_v, seg_v, sem):
    cid = lax.axis_index("c")
    sid = lax.axis_index("s")
    w = cid * 16 + sid
    pltpu.sync_copy(batch_h, batch_v)
    pltpu.sync_copy(ip_h.at[pl.ds(w * NI_W, NI_W)], ip_v)
    pltpu.async_copy(xjk_h.at[ip_v], rows_v, sem).wait()
    pltpu.sync_copy(rows_v, xi_h.at[pl.ds(w * NI_W, NI_W)])
    for k in range(NI_W // 16):
        iv = ip_v[pl.ds(k * 16, 16)]
        sg = plsc.load_gather(batch_v, [iv])
        seg_v[pl.ds(k * 16, 16)] = sg.astype(jnp.float32)
    pltpu.sync_copy(seg_v, seg_h.at[pl.ds(w * NI_W, NI_W)])


def _sc_gather(xjk, ipos, batchp):
    mesh = plsc.VectorSubcoreMesh(core_axis_name="c", subcore_axis_name="s",
                                  num_cores=2, num_subcores=16)
    f = pl.kernel(
        _sc_gather_body,
        out_type=[
            jax.ShapeDtypeStruct((NI, 128), jnp.float32),
            jax.ShapeDtypeStruct((NI,), jnp.float32),
        ],
        mesh=mesh,
        compiler_params=pltpu.CompilerParams(needs_layout_passes=False),
        scratch_types=[
            pltpu.VMEM((NP,), jnp.int32),
            pltpu.VMEM((NI_W,), jnp.int32),
            pltpu.VMEM((NI_W, 128), jnp.float32),
            pltpu.VMEM((NI_W,), jnp.float32),
            pltpu.SemaphoreType.DMA,
        ],
    )
    return f(xjk, ipos, batchp)


def _final_body(xi_ref, seg_ref, gw_ref, l1w_ref, l1b_ref, l2w_ref, l2b_ref,
                out_ref):
    xi = xi_ref[...]
    seg = seg_ref[...]
    segb = jnp.broadcast_to(seg[None, :], (G, NI))
    gi = lax.broadcasted_iota(jnp.int32, (G, NI), 0).astype(jnp.float32)
    oh = (segb == gi).astype(jnp.float32)
    addp = jnp.dot(oh, xi, preferred_element_type=jnp.float32)
    cnt = jnp.sum(oh, axis=1)
    meanp = addp / jnp.maximum(cnt, 1.0)[:, None]
    sa = jnp.sum(xi * gw_ref[...], axis=1)
    neg = jnp.float32(-jnp.inf)
    sab = jnp.broadcast_to(sa[None, :], (G, NI))
    mg = jnp.max(jnp.where(oh > 0, sab, neg), axis=1)
    mgc = jnp.maximum(mg, -1e30)
    mrow = jnp.sum(oh * mgc[:, None], axis=0)
    eatt = jnp.exp(sa - mrow)
    ssum = jnp.sum(oh * eatt[None, :], axis=1)
    attp = jnp.dot(oh, xi * eatt[:, None], preferred_element_type=jnp.float32)
    attp = jnp.where((ssum > 0)[:, None],
                     attp / jnp.maximum(ssum, 1e-30)[:, None], 0.0)
    mxs = []
    for g in range(G):
        row = oh[g][:, None] > 0
        mxs.append(jnp.max(jnp.where(row, xi, neg), axis=0, keepdims=True))
    maxp = jnp.concatenate(mxs, axis=0)
    xc = jnp.concatenate([addp, meanp, attp, maxp], axis=1)
    h = jnp.tanh(jnp.dot(xc, l1w_ref[...], preferred_element_type=jnp.float32)
                 + l1b_ref[...])
    out_ref[...] = jnp.sum(h * l2w_ref[...], axis=1)[:, None] + l2b_ref[...]


def _tc_final(xi, segf, gw, l1wT, l1b2, l2w2, l2b2):
    return pl.pallas_call(
        _final_body,
        out_shape=jax.ShapeDtypeStruct((G, 128), jnp.float32),
    )(xi, segf, gw, l1wT, l1b2, l2w2, l2b2)


def kernel(x, edge_index, edge_attr, batch, interface_pos, graph_num,
           W0, attL0, eW0, eb0, b0,
           W1, attL1, eW1, eb1, b1,
           W2, attL2, eW2, eb2, b2,
           gate_w, lin1_W, lin1_b, lin2_W, lin2_b):
    ar = jnp.arange(N, dtype=jnp.int32)
    src = jnp.concatenate([edge_index[0].astype(jnp.int32), ar])
    dst = jnp.concatenate([edge_index[1].astype(jnp.int32), ar])
    zpad = jnp.zeros((EPAD - EE,), jnp.int32)
    srcp = jnp.concatenate([src, zpad]).reshape(NT, NB, R)
    dstp = jnp.concatenate([dst, zpad]).reshape(NT, NB, R)
    ea_full = jnp.concatenate([
        edge_attr.astype(jnp.float32),
        jnp.zeros((N, DE), jnp.float32),
        jnp.ones((EPAD - EE, DE), jnp.float32),
    ], axis=0)
    eat = ea_full.reshape(NT, CHUNK, DE).transpose(0, 2, 1).reshape(NT, 4 * CHUNK)

    xcur = jnp.pad(x.astype(jnp.float32), ((0, NP - N), (0, 0)))
    batchp = jnp.pad(batch.astype(jnp.int32), (0, NP - N))

    layers = [(W0, attL0, eW0, eb0, b0),
              (W1, attL1, eW1, eb1, b1),
              (W2, attL2, eW2, eb2, b2)]
    outs = []
    xjk = None
    for li, (W, attL, eW, eb, b) in enumerate(layers):
        xl, al2, m8 = _tc_pre(xcur, W, attL.reshape(1, C))
        sparts, tparts, e_h = _sc_p1(srcp, dstp, eat,
                                     al2.reshape(-1), m8.reshape(-1))
        acc = _sc_p2(srcp, dstp, e_h, xl)
        ewT = eW.T
        ebb = (eb + b).reshape(1, C)
        if li < 2:
            (xcur,) = _tc_post(sparts, tparts.reshape(NT, 4, NP), acc, ewT, ebb)
            outs.append(xcur)
        else:
            _, xjk = _tc_post(sparts, tparts.reshape(NT, 4, NP), acc, ewT, ebb,
                              outs[0], outs[1])

    xi, segf = _sc_gather(xjk, interface_pos.astype(jnp.int32), batchp)
    out128 = _tc_final(xi, segf,
                       gate_w.reshape(1, C),
                       lin1_W.T,
                       lin1_b.reshape(1, 2 * C),
                       lin2_W.reshape(1, 2 * C),
                       jnp.broadcast_to(lin2_b.reshape(1, 1), (1, 128)))
    return out128[:, :1]
